# baseline (speedup 1.0000x reference)
import dataclasses
import functools

import jax
import jax.numpy as jnp
from jax import lax
from jax.experimental import pallas as pl
from jax.experimental.pallas import tpu as pltpu
from jax.experimental.pallas import tpu_sc as plsc

E = 160000
D = 256
NC, NS, L = 2, 16, 16
NW = NC * NS
EPW = E // NW
C = 64
G = -(-EPW // C)
LAST = EPW - C


def _sc_edge_dot(x, senders, receivers):
    mesh = plsc.VectorSubcoreMesh(core_axis_name="c", subcore_axis_name="s")
    cp = pltpu.CompilerParams()
    if "needs_layout_passes" in pltpu.CompilerParams.__dataclass_fields__:
        cp = dataclasses.replace(cp, needs_layout_passes=False)
    if "use_tc_tiling_on_sc" in pltpu.CompilerParams.__dataclass_fields__:
        cp = dataclasses.replace(cp, use_tc_tiling_on_sc=False)

    @functools.partial(
        pl.kernel,
        compiler_params=cp,
        out_type=jax.ShapeDtypeStruct((E,), jnp.float32),
        mesh=mesh,
        scratch_types=[
            pltpu.VMEM((EPW,), jnp.int32),
            pltpu.VMEM((EPW,), jnp.int32),
            [pltpu.VMEM((C, D), jnp.float32) for _ in range(2)],
            [pltpu.VMEM((C, D), jnp.float32) for _ in range(2)],
            pltpu.VMEM((EPW,), jnp.float32),
            [pltpu.SemaphoreType.DMA for _ in range(2)],
        ],
    )
    def k(x_hbm, s_hbm, r_hbm, o_hbm, s_v, r_v, xs_v, xr_v, o_v, sem):
        wid = lax.axis_index("s") * NC + lax.axis_index("c")
        ebase = wid * EPW
        iota16 = lax.iota(jnp.int32, L)

        pltpu.sync_copy(s_hbm.at[pl.ds(ebase, EPW)], s_v)
        pltpu.sync_copy(r_hbm.at[pl.ds(ebase, EPW)], r_v)

        def chunk_off(g):
            return jnp.minimum(g * C, LAST)

        def issue(g, b):
            off = chunk_off(g)
            pltpu.async_copy(x_hbm.at[s_v.at[pl.ds(off, C)]], xs_v[b], sem[b])
            pltpu.async_copy(x_hbm.at[r_v.at[pl.ds(off, C)]], xr_v[b], sem[b])

        def compute(g, b):
            off = chunk_off(g)
            pltpu.make_async_copy(
                x_hbm.at[s_v.at[pl.ds(off, C)]], xs_v[b], sem[b]).wait()
            pltpu.make_async_copy(
                x_hbm.at[r_v.at[pl.ds(off, C)]], xr_v[b], sem[b]).wait()

            @pl.loop(0, C, step=L)
            def _(e0):
                rows = iota16 + e0

                @pl.loop(0, D, init_carry=jnp.zeros((L,), jnp.float32),
                         unroll=32)
                def acc(d, a):
                    cols = iota16 + d
                    cols = jnp.where(cols >= D, cols - D, cols)
                    xs = plsc.load_gather(xs_v[b], [rows, cols])
                    xr = plsc.load_gather(xr_v[b], [rows, cols])
                    return a + xs * xr

                o_v[pl.ds(off + e0, L)] = acc

        issue(0, 0)

        @pl.loop(0, G + 1, step=2)
        def _(g):
            pl.when(g + 1 < G)(lambda: issue(g + 1, 1))
            compute(g, 0)
            pl.when(g + 2 < G)(lambda: issue(g + 2, 0))
            pl.when(g + 1 < G)(lambda: compute(g + 1, 1))

        pltpu.sync_copy(o_v, o_hbm.at[pl.ds(ebase, EPW)])

    return k(x, senders, receivers)


def kernel(x, edge_index):
    senders = edge_index[0].astype(jnp.int32)
    receivers = edge_index[1].astype(jnp.int32)
    he = _sc_edge_dot(x, senders, receivers)
    return he.reshape(E, 1)


# speedup vs baseline: 1.4016x; 1.4016x over previous
import dataclasses
import functools

import jax
import jax.numpy as jnp
from jax import lax
from jax.experimental import pallas as pl
from jax.experimental.pallas import tpu as pltpu
from jax.experimental.pallas import tpu_sc as plsc

E = 160000
D = 256
NC, NS, L = 2, 16, 16
NW = NC * NS
EPW = E // NW
C = 64
G = -(-EPW // C)
LAST = EPW - C


def _sc_edge_dot(x, senders, receivers):
    mesh = plsc.VectorSubcoreMesh(core_axis_name="c", subcore_axis_name="s")
    cp = pltpu.CompilerParams()
    if "needs_layout_passes" in pltpu.CompilerParams.__dataclass_fields__:
        cp = dataclasses.replace(cp, needs_layout_passes=False)
    if "use_tc_tiling_on_sc" in pltpu.CompilerParams.__dataclass_fields__:
        cp = dataclasses.replace(cp, use_tc_tiling_on_sc=False)

    @functools.partial(
        pl.kernel,
        compiler_params=cp,
        out_type=jax.ShapeDtypeStruct((E,), jnp.float32),
        mesh=mesh,
        scratch_types=[
            pltpu.VMEM((EPW,), jnp.int32),
            pltpu.VMEM((EPW,), jnp.int32),
            [pltpu.VMEM((C, D), jnp.float32) for _ in range(2)],
            [pltpu.VMEM((C, D), jnp.float32) for _ in range(2)],
            pltpu.VMEM((EPW,), jnp.float32),
            [pltpu.SemaphoreType.DMA for _ in range(2)],
        ],
    )
    def k(x_hbm, s_hbm, r_hbm, o_hbm, s_v, r_v, xs_v, xr_v, o_v, sem):
        wid = lax.axis_index("s") * NC + lax.axis_index("c")
        ebase = wid * EPW
        iota16 = lax.iota(jnp.int32, L)

        pltpu.sync_copy(s_hbm.at[pl.ds(ebase, EPW)], s_v)
        pltpu.sync_copy(r_hbm.at[pl.ds(ebase, EPW)], r_v)

        def chunk_off(g):
            return jnp.minimum(g * C, LAST)

        def issue(g, b):
            off = chunk_off(g)
            pltpu.async_copy(x_hbm.at[s_v.at[pl.ds(off, C)]], xs_v[b], sem[b])
            pltpu.async_copy(x_hbm.at[r_v.at[pl.ds(off, C)]], xr_v[b], sem[b])

        def compute(g, b):
            off = chunk_off(g)
            pltpu.make_async_copy(
                x_hbm.at[s_v.at[pl.ds(off, C)]], xs_v[b], sem[b]).wait()
            pltpu.make_async_copy(
                x_hbm.at[r_v.at[pl.ds(off, C)]], xr_v[b], sem[b]).wait()

            @pl.loop(0, C, step=L)
            def _(e0):
                rows = iota16 + e0

                @pl.loop(0, D, init_carry=jnp.zeros((L,), jnp.float32),
                         unroll=8)
                def acc(d, a):
                    cols = iota16 + d
                    cols = jnp.where(cols >= D, cols - D, cols)
                    xs = plsc.load_gather(xs_v[b], [rows, cols])
                    xr = plsc.load_gather(xr_v[b], [rows, cols])
                    return a + xs * xr

                o_v[pl.ds(off + e0, L)] = acc

        issue(0, 0)

        @pl.loop(0, G + 1, step=2)
        def _(g):
            pl.when(g + 1 < G)(lambda: issue(g + 1, 1))
            compute(g, 0)
            pl.when(g + 2 < G)(lambda: issue(g + 2, 0))
            pl.when(g + 1 < G)(lambda: compute(g + 1, 1))

        pltpu.sync_copy(o_v, o_hbm.at[pl.ds(ebase, EPW)])

    return k(x, senders, receivers)


def kernel(x, edge_index):
    senders = edge_index[0].astype(jnp.int32)
    receivers = edge_index[1].astype(jnp.int32)
    he = _sc_edge_dot(x, senders, receivers)
    return he.reshape(E, 1)


# speedup vs baseline: 1.6006x; 1.1420x over previous
import dataclasses
import functools

import jax
import jax.numpy as jnp
from jax import lax
from jax.experimental import pallas as pl
from jax.experimental.pallas import tpu as pltpu
from jax.experimental.pallas import tpu_sc as plsc

E = 160000
D = 256
NC, NS, L = 2, 16, 16
NW = NC * NS
EPW = E // NW
C = 64
G = -(-EPW // C)
LAST = EPW - C


def _sc_edge_dot(x, senders, receivers):
    mesh = plsc.VectorSubcoreMesh(core_axis_name="c", subcore_axis_name="s")
    cp = pltpu.CompilerParams()
    if "needs_layout_passes" in pltpu.CompilerParams.__dataclass_fields__:
        cp = dataclasses.replace(cp, needs_layout_passes=False)
    if "use_tc_tiling_on_sc" in pltpu.CompilerParams.__dataclass_fields__:
        cp = dataclasses.replace(cp, use_tc_tiling_on_sc=False)

    @functools.partial(
        pl.kernel,
        compiler_params=cp,
        out_type=jax.ShapeDtypeStruct((E,), jnp.float32),
        mesh=mesh,
        scratch_types=[
            pltpu.VMEM((EPW,), jnp.int32),
            pltpu.VMEM((EPW,), jnp.int32),
            [pltpu.VMEM((C, D), jnp.float32) for _ in range(2)],
            [pltpu.VMEM((C, D), jnp.float32) for _ in range(2)],
            pltpu.VMEM((EPW,), jnp.float32),
            [pltpu.SemaphoreType.DMA for _ in range(2)],
        ],
    )
    def k(x_hbm, s_hbm, r_hbm, o_hbm, s_v, r_v, xs_v, xr_v, o_v, sem):
        wid = lax.axis_index("s") * NC + lax.axis_index("c")
        ebase = wid * EPW
        iota16 = lax.iota(jnp.int32, L)

        pltpu.sync_copy(s_hbm.at[pl.ds(ebase, EPW)], s_v)
        pltpu.sync_copy(r_hbm.at[pl.ds(ebase, EPW)], r_v)

        def chunk_off(g):
            return jnp.minimum(g * C, LAST)

        def issue(g, b):
            off = chunk_off(g)
            pltpu.async_copy(x_hbm.at[s_v.at[pl.ds(off, C)]], xs_v[b], sem[b])
            pltpu.async_copy(x_hbm.at[r_v.at[pl.ds(off, C)]], xr_v[b], sem[b])

        def compute(g, b):
            off = chunk_off(g)
            pltpu.make_async_copy(
                x_hbm.at[s_v.at[pl.ds(off, C)]], xs_v[b], sem[b]).wait()
            pltpu.make_async_copy(
                x_hbm.at[r_v.at[pl.ds(off, C)]], xr_v[b], sem[b]).wait()

            @pl.loop(0, C, step=L)
            def _(e0):
                rows = iota16 + e0

                @pl.loop(0, D - L, init_carry=jnp.zeros((L,), jnp.float32),
                         unroll=16)
                def acc(d, a):
                    cols = iota16 + d
                    xs = plsc.load_gather(xs_v[b], [rows, cols])
                    xr = plsc.load_gather(xr_v[b], [rows, cols])
                    return a + xs * xr

                @pl.loop(D - L, D, init_carry=acc, unroll=16)
                def acc2(d, a):
                    cols = iota16 + d
                    cols = jnp.where(cols >= D, cols - D, cols)
                    xs = plsc.load_gather(xs_v[b], [rows, cols])
                    xr = plsc.load_gather(xr_v[b], [rows, cols])
                    return a + xs * xr

                o_v[pl.ds(off + e0, L)] = acc2

        issue(0, 0)

        @pl.loop(0, G + 1, step=2)
        def _(g):
            pl.when(g + 1 < G)(lambda: issue(g + 1, 1))
            compute(g, 0)
            pl.when(g + 2 < G)(lambda: issue(g + 2, 0))
            pl.when(g + 1 < G)(lambda: compute(g + 1, 1))

        pltpu.sync_copy(o_v, o_hbm.at[pl.ds(ebase, EPW)])

    return k(x, senders, receivers)


def kernel(x, edge_index):
    senders = edge_index[0].astype(jnp.int32)
    receivers = edge_index[1].astype(jnp.int32)
    he = _sc_edge_dot(x, senders, receivers)
    return he.reshape(E, 1)
